# Initial kernel scaffold; baseline (speedup 1.0000x reference)
import functools

import jax
import jax.numpy as jnp
from jax import lax
from jax.experimental import pallas as pl
from jax.experimental.pallas import tpu as pltpu
from jax.experimental.pallas import tpu_sc as plsc

B, S, D = 4096, 200, 64
N = B * S
V = 100000
PAD = 1
NC, NS, L = 2, 16, 16
NW = NC * NS
TOK_PER_W = N // NW
CHUNK = 1024
NCHUNK = TOK_PER_W // CHUNK
GROW = 128
G = CHUNK // GROW

_mesh = plsc.VectorSubcoreMesh(core_axis_name="c", subcore_axis_name="s")


@functools.partial(
    pl.kernel,
    mesh=_mesh,
    out_type=jax.ShapeDtypeStruct((N, D), jnp.float32),
    scratch_types=[
        pltpu.VMEM((CHUNK,), jnp.int32),
        pltpu.VMEM((CHUNK,), jnp.int32),
        pltpu.VMEM((G, GROW), jnp.int32),
        pltpu.VMEM((CHUNK, D), jnp.float32),
        pltpu.SemaphoreType.DMA,
    ],
)
def _sc_gather(concept_hbm, ttype_hbm, table_hbm, out_hbm,
               c_v, t_v, idx_v, rows_v, sem):
    wid = lax.axis_index("s") * NC + lax.axis_index("c")

    def chunk_body(ci, carry):
        base = pl.multiple_of(wid * TOK_PER_W + ci * CHUNK, CHUNK)
        pltpu.sync_copy(concept_hbm.at[pl.ds(base, CHUNK)], c_v)
        pltpu.sync_copy(ttype_hbm.at[pl.ds(base, CHUNK)], t_v)
        for j in range(G):
            for q in range(GROW // L):
                o = j * GROW + q * L
                c = c_v[pl.ds(o, L)]
                t = t_v[pl.ds(o, L)]
                fi = jnp.where(
                    t == 1, c,
                    jnp.where(t == 2, c + V,
                              jnp.where(t == 3, c + 2 * V,
                                        jnp.full((L,), PAD, jnp.int32))))
                idx_v[j, pl.ds(q * L, L)] = fi
        copies = [
            pltpu.async_copy(table_hbm.at[idx_v.at[j]],
                             rows_v.at[pl.ds(j * GROW, GROW)], sem)
            for j in range(G)
        ]
        for cp in copies:
            cp.wait()
        pltpu.sync_copy(rows_v, out_hbm.at[pl.ds(base, CHUNK)])
        return carry

    lax.fori_loop(0, NCHUNK, chunk_body, 0)


def kernel(concept, token_type, proc_table, med_table, chart_table):
    table = jnp.concatenate([proc_table, med_table, chart_table], axis=0)
    c = concept.reshape(N).astype(jnp.int32)
    t = token_type.reshape(N).astype(jnp.int32)
    out = _sc_gather(c, t, table)
    return out.reshape(B, S, D)


# speedup vs baseline: 1.6668x; 1.6668x over previous
import functools

import jax
import jax.numpy as jnp
from jax import lax
from jax.experimental import pallas as pl
from jax.experimental.pallas import tpu as pltpu
from jax.experimental.pallas import tpu_sc as plsc

B, S, D = 4096, 200, 64
N = B * S
V = 100000
PAD = 1
NC, NS, L = 2, 16, 16
NW = NC * NS
TOK_PER_W = N // NW
CHUNK = 1024
NCHUNK = TOK_PER_W // CHUNK
GROW = 128
G = CHUNK // GROW

_mesh = plsc.VectorSubcoreMesh(core_axis_name="c", subcore_axis_name="s")


@functools.partial(
    pl.kernel,
    mesh=_mesh,
    out_type=jax.ShapeDtypeStruct((N, D), jnp.float32),
    scratch_types=[
        pltpu.VMEM((CHUNK,), jnp.int32),
        pltpu.VMEM((CHUNK,), jnp.int32),
        pltpu.VMEM((G, GROW), jnp.int32),
        pltpu.VMEM((CHUNK, D), jnp.float32),
        pltpu.SemaphoreType.DMA,
    ],
    compiler_params=pltpu.CompilerParams(use_tc_tiling_on_sc=False),
)
def _sc_gather(concept_hbm, ttype_hbm, table_hbm, out_hbm,
               c_v, t_v, idx_v, rows_v, sem):
    wid = lax.axis_index("s") * NC + lax.axis_index("c")

    def chunk_body(ci, carry):
        base = pl.multiple_of(wid * TOK_PER_W + ci * CHUNK, CHUNK)
        pltpu.sync_copy(concept_hbm.at[pl.ds(base, CHUNK)], c_v)
        pltpu.sync_copy(ttype_hbm.at[pl.ds(base, CHUNK)], t_v)
        for j in range(G):
            for q in range(GROW // L):
                o = j * GROW + q * L
                c = c_v[pl.ds(o, L)]
                t = t_v[pl.ds(o, L)]
                fi = jnp.where(
                    t == 1, c,
                    jnp.where(t == 2, c + V,
                              jnp.where(t == 3, c + 2 * V,
                                        jnp.full((L,), PAD, jnp.int32))))
                idx_v[j, pl.ds(q * L, L)] = fi
        copies = [
            pltpu.async_copy(table_hbm.at[idx_v.at[j]],
                             rows_v.at[pl.ds(j * GROW, GROW)], sem)
            for j in range(G)
        ]
        for cp in copies:
            cp.wait()
        pltpu.sync_copy(rows_v, out_hbm.at[pl.ds(base, CHUNK)])
        return carry

    lax.fori_loop(0, NCHUNK, chunk_body, 0)


def kernel(concept, token_type, proc_table, med_table, chart_table):
    table = jnp.concatenate([proc_table, med_table, chart_table], axis=0)
    c = concept.reshape(N).astype(jnp.int32)
    t = token_type.reshape(N).astype(jnp.int32)
    out = _sc_gather(c, t, table)
    return out.reshape(B, S, D)


# speedup vs baseline: 7.9927x; 4.7952x over previous
import functools

import jax
import jax.numpy as jnp
from jax import lax
from jax.experimental import pallas as pl
from jax.experimental.pallas import tpu as pltpu
from jax.experimental.pallas import tpu_sc as plsc

B, S, D = 4096, 200, 64
N = B * S
V = 100000
Z = 8192
NC, NS, L = 2, 16, 16
NW = NC * NS
TOK_PER_W = N // NW
CHUNK = 640
NCHUNK = TOK_PER_W // CHUNK
PAIRS = NCHUNK // 2
GROW = 128
G = CHUNK // GROW

_mesh = plsc.VectorSubcoreMesh(core_axis_name="c", subcore_axis_name="s")


@functools.partial(
    pl.kernel,
    mesh=_mesh,
    out_type=jax.ShapeDtypeStruct((N, D), jnp.float32),
    scratch_types=[
        pltpu.VMEM((2, CHUNK), jnp.int32),
        pltpu.VMEM((2, CHUNK), jnp.int32),
        pltpu.VMEM((2, G, GROW), jnp.int32),
        pltpu.VMEM((CHUNK, D), jnp.float32),
        pltpu.VMEM((CHUNK, D), jnp.float32),
        pltpu.SemaphoreType.DMA,
        pltpu.SemaphoreType.DMA,
    ],
    compiler_params=pltpu.CompilerParams(use_tc_tiling_on_sc=False),
)
def _sc_gather(concept_hbm, ttype_hbm, table_hbm, out_hbm,
               c_v, t_v, idx_v, rows0_v, rows1_v, gsem0, gsem1):
    wid = lax.axis_index("s") * NC + lax.axis_index("c")
    w_base = wid * TOK_PER_W
    rows = (rows0_v, rows1_v)
    gsem = (gsem0, gsem1)

    def prep(ci, b):
        base = pl.multiple_of(w_base + ci * CHUNK, CHUNK)
        pltpu.sync_copy(concept_hbm.at[pl.ds(base, CHUNK)], c_v.at[b])
        pltpu.sync_copy(ttype_hbm.at[pl.ds(base, CHUNK)], t_v.at[b])
        for j in range(G):
            for q in range(GROW // L):
                o = j * GROW + q * L
                c = c_v[b, pl.ds(o, L)]
                t = t_v[b, pl.ds(o, L)]
                fi = jnp.where(
                    t == 1, c,
                    jnp.where(t == 2, c + V,
                              jnp.where(t == 3, c + 2 * V,
                                        3 * V + (c & (Z - 1)))))
                idx_v[b, j, pl.ds(q * L, L)] = fi

    def fire(b):
        for j in range(G):
            pltpu.async_copy(table_hbm.at[idx_v.at[b, j]],
                             rows[b].at[pl.ds(j * GROW, GROW)], gsem[b])

    def drain(b):
        for j in range(G):
            pltpu.make_async_copy(table_hbm.at[idx_v.at[b, j]],
                                  rows[b].at[pl.ds(j * GROW, GROW)],
                                  gsem[b]).wait()

    def writeback(ci, b):
        base = pl.multiple_of(w_base + ci * CHUNK, CHUNK)
        pltpu.sync_copy(rows[b], out_hbm.at[pl.ds(base, CHUNK)])

    prep(0, 0)
    fire(0)

    def pair_body(p, carry):
        i = 2 * p
        prep(i + 1, 1)
        fire(1)
        drain(0)
        writeback(i, 0)
        @pl.when(p + 1 < PAIRS)
        def _():
            prep(i + 2, 0)
            fire(0)
        drain(1)
        writeback(i + 1, 1)
        return carry

    lax.fori_loop(0, PAIRS, pair_body, 0)


def kernel(concept, token_type, proc_table, med_table, chart_table):
    table = jnp.concatenate(
        [proc_table, med_table, chart_table,
         jnp.zeros((Z, D), jnp.float32)], axis=0)
    c = concept.reshape(N).astype(jnp.int32)
    t = token_type.reshape(N).astype(jnp.int32)
    out = _sc_gather(c, t, table)
    return out.reshape(B, S, D)


# speedup vs baseline: 9.4867x; 1.1869x over previous
import functools

import jax
import jax.numpy as jnp
from jax import lax
from jax.experimental import pallas as pl
from jax.experimental.pallas import tpu as pltpu
from jax.experimental.pallas import tpu_sc as plsc

B, S, D = 4096, 200, 64
N = B * S
V = 100000
Z = 8192
NC, NS, L = 2, 16, 16
NW = NC * NS
TOK_PER_W = N // NW
BROWS = 4
CHUNK = BROWS * S
NCHUNK = TOK_PER_W // CHUNK
PAIRS = NCHUNK // 2
GROW = 40
G = CHUNK // GROW
SPB = S // GROW

_mesh = plsc.VectorSubcoreMesh(core_axis_name="c", subcore_axis_name="s")

VTOT = 3 * V + Z
ROWS_PER_W = V // NW
FCHUNK = 625
NFSTEP = 3 * (ROWS_PER_W // FCHUNK)
ZROWS_PER_W = Z // NW


@functools.partial(
    pl.kernel,
    mesh=_mesh,
    out_type=jax.ShapeDtypeStruct((VTOT, D), jnp.float32),
    scratch_types=[
        pltpu.VMEM((FCHUNK, D), jnp.float32),
        pltpu.VMEM((FCHUNK, D), jnp.float32),
        pltpu.SemaphoreType.DMA,
        pltpu.SemaphoreType.DMA,
        pltpu.SemaphoreType.DMA,
        pltpu.SemaphoreType.DMA,
    ],
    compiler_params=pltpu.CompilerParams(use_tc_tiling_on_sc=False),
)
def _sc_fuse(proc_hbm, med_hbm, chart_hbm, fused_hbm, buf0, buf1,
             isem0, isem1, osem0, osem1):
    wid = lax.axis_index("s") * NC + lax.axis_index("c")
    bufs = (buf0, buf1)
    isem = (isem0, isem1)
    osem = (osem0, osem1)
    tables = (proc_hbm, med_hbm, chart_hbm)
    steps_per_table = ROWS_PER_W // FCHUNK

    def src_dst(step):
        tab = step // steps_per_table
        k = step % steps_per_table
        row = wid * ROWS_PER_W + k * FCHUNK
        return tables[tab].at[pl.ds(row, FCHUNK)], \
            fused_hbm.at[pl.ds(tab * V + row, FCHUNK)]

    def fire_in(step, b):
        s, _ = src_dst(step)
        pltpu.async_copy(s, bufs[b], isem[b])

    def wait_in(step, b):
        s, _ = src_dst(step)
        pltpu.make_async_copy(s, bufs[b], isem[b]).wait()

    def fire_out(step, b):
        _, d = src_dst(step)
        pltpu.async_copy(bufs[b], d, osem[b])

    def wait_out(step, b):
        _, d = src_dst(step)
        pltpu.make_async_copy(bufs[b], d, osem[b]).wait()

    fire_in(0, 0)
    for step in range(NFSTEP):
        b = step % 2
        if step + 1 < NFSTEP:
            if step >= 1:
                wait_out(step - 1, 1 - b)
            fire_in(step + 1, 1 - b)
        wait_in(step, b)
        fire_out(step, b)
    wait_out(NFSTEP - 2, (NFSTEP - 2) % 2)
    wait_out(NFSTEP - 1, (NFSTEP - 1) % 2)
    zero = jnp.zeros((L,), jnp.float32)

    def zbody(r, carry):
        for q in range(D // L):
            buf0[r, pl.ds(q * L, L)] = zero
        return carry

    lax.fori_loop(0, ZROWS_PER_W, zbody, 0)
    zdst = fused_hbm.at[pl.ds(3 * V + wid * ZROWS_PER_W, ZROWS_PER_W)]
    pltpu.sync_copy(buf0.at[pl.ds(0, ZROWS_PER_W)], zdst)


@functools.partial(
    pl.kernel,
    mesh=_mesh,
    out_type=jax.ShapeDtypeStruct((B, S, D), jnp.float32),
    scratch_types=[
        pltpu.VMEM((2, CHUNK), jnp.int32),
        pltpu.VMEM((2, CHUNK), jnp.int32),
        pltpu.VMEM((2, CHUNK), jnp.int32),
        pltpu.VMEM((BROWS, S, D), jnp.float32),
        pltpu.VMEM((BROWS, S, D), jnp.float32),
        pltpu.SemaphoreType.DMA,
        pltpu.SemaphoreType.DMA,
    ],
    compiler_params=pltpu.CompilerParams(use_tc_tiling_on_sc=False),
)
def _sc_gather(concept_hbm, ttype_hbm, table_hbm, out_hbm,
               c_v, t_v, idx_v, rows0_v, rows1_v, gsem0, gsem1):
    wid = lax.axis_index("s") * NC + lax.axis_index("c")
    w_base = wid * TOK_PER_W
    rows = (rows0_v, rows1_v)
    gsem = (gsem0, gsem1)

    def prep(ci, b):
        base = pl.multiple_of(w_base + ci * CHUNK, CHUNK)
        pltpu.sync_copy(concept_hbm.at[pl.ds(base, CHUNK)], c_v.at[b])
        pltpu.sync_copy(ttype_hbm.at[pl.ds(base, CHUNK)], t_v.at[b])
        for q in range(CHUNK // L):
            o = q * L
            c = c_v[b, pl.ds(o, L)]
            t = t_v[b, pl.ds(o, L)]
            fi = jnp.where(
                t == 1, c,
                jnp.where(t == 2, c + V,
                          jnp.where(t == 3, c + 2 * V,
                                    3 * V + (c & (Z - 1)))))
            idx_v[b, pl.ds(o, L)] = fi

    def stream_parts(b):
        for j in range(G):
            yield (idx_v.at[b, pl.ds(j * GROW, GROW)],
                   rows[b].at[j // SPB, pl.ds((j % SPB) * GROW, GROW), :])

    def fire(b):
        for idx_s, dst in stream_parts(b):
            pltpu.async_copy(table_hbm.at[idx_s], dst, gsem[b])

    def drain(b):
        for idx_s, dst in stream_parts(b):
            pltpu.make_async_copy(table_hbm.at[idx_s], dst, gsem[b]).wait()

    def writeback(ci, b):
        b0 = pl.multiple_of(wid * (B // NW) + ci * BROWS, BROWS)
        pltpu.sync_copy(rows[b], out_hbm.at[pl.ds(b0, BROWS), :, :])

    prep(0, 0)
    fire(0)

    def pair_body(p, carry):
        i = 2 * p
        prep(i + 1, 1)
        fire(1)
        drain(0)
        writeback(i, 0)
        @pl.when(p + 1 < PAIRS)
        def _():
            prep(i + 2, 0)
            fire(0)
        drain(1)
        writeback(i + 1, 1)
        return carry

    lax.fori_loop(0, PAIRS, pair_body, 0)


def kernel(concept, token_type, proc_table, med_table, chart_table):
    table = _sc_fuse(proc_table, med_table, chart_table)
    c = concept.reshape(N).astype(jnp.int32)
    t = token_type.reshape(N).astype(jnp.int32)
    return _sc_gather(c, t, table)
